# Initial kernel scaffold; baseline (speedup 1.0000x reference)
import jax
import jax.numpy as jnp
from jax.experimental import pallas as pl


def kernel(input_len, table):
    raise NotImplementedError("write your pallas kernel here")


# speedup vs baseline: 17.0153x; 17.0153x over previous
import functools

import jax
import jax.numpy as jnp
from jax import lax
from jax.experimental import pallas as pl
from jax.experimental.pallas import tpu as pltpu
from jax.experimental.pallas import tpu_sc as plsc

MODEL_DIM = 128
MAX_LEN = 200
BATCH = 4096

NC = 2
NS = 16
NW = NC * NS
RPW = BATCH // NW
POS_W = 208
BITS = (128, 64, 32, 16, 8, 4, 2, 1)

_mesh = plsc.VectorSubcoreMesh(
    core_axis_name="c", subcore_axis_name="s", num_cores=NC, num_subcores=NS)


@functools.partial(
    pl.kernel,
    out_type=(
        jax.ShapeDtypeStruct((BATCH, MAX_LEN, MODEL_DIM), jnp.float32),
        jax.ShapeDtypeStruct((BATCH, MAX_LEN), jnp.int32),
    ),
    mesh=_mesh,
    compiler_params=pltpu.CompilerParams(use_tc_tiling_on_sc=False),
    scratch_types=[
        pltpu.VMEM((MAX_LEN, MODEL_DIM), jnp.float32),
        pltpu.VMEM((128, MODEL_DIM), jnp.float32),
        pltpu.VMEM((RPW,), jnp.int32),
        pltpu.VMEM((RPW, POS_W), jnp.int32),
        pltpu.SemaphoreType.DMA,
    ],
)
def _pe_kernel(len_hbm, table_hbm, emb_hbm, pos_hbm,
               tbl_v, zero_v, lens_v, pos_v, sem):
    wid = lax.axis_index("s") * NC + lax.axis_index("c")
    base = wid * RPW

    pltpu.sync_copy(table_hbm.at[pl.ds(1, MAX_LEN)], tbl_v)
    pltpu.sync_copy(len_hbm.at[pl.ds(base, RPW)], lens_v)

    zvec = jnp.zeros((16,), jnp.float32)

    def _zero_row(r, _):
        for c in range(MODEL_DIM // 16):
            zero_v[r, pl.ds(c * 16, 16)] = zvec
        return 0

    lax.fori_loop(0, 128, _zero_row, 0)

    iota = lax.iota(jnp.int32, 16)

    def _blk(blk, _):
        lv = lens_v[pl.ds(blk * 16, 16)]
        for lane in range(16):
            L = lv[lane]
            r = blk * 16 + lane
            row = base + r
            tail = MAX_LEN - L
            for b in BITS:
                off = L & ~(2 * b - 1)

                @pl.when((L & b) != 0)
                def _():
                    pltpu.async_copy(
                        tbl_v.at[pl.ds(off, b)],
                        emb_hbm.at[row, pl.ds(off, b)],
                        sem,
                    )
            for b in BITS:
                zoff = L + (tail & ~(2 * b - 1))

                @pl.when((tail & b) != 0)
                def _():
                    pltpu.async_copy(
                        zero_v.at[pl.ds(0, b)],
                        emb_hbm.at[row, pl.ds(zoff, b)],
                        sem,
                    )
            splat = jnp.full((16,), L, jnp.int32)
            for c in range(POS_W // 16):
                vec = iota + (16 * c + 1)
                pos_v[r, pl.ds(16 * c, 16)] = jnp.where(vec <= splat, vec, 0)
        return 0

    lax.fori_loop(0, RPW // 16, _blk, 0)

    def _drain(r, _):
        pltpu.make_async_copy(
            table_hbm.at[pl.ds(1, MAX_LEN)], tbl_v, sem).wait()
        return 0

    lax.fori_loop(0, RPW, _drain, 0)

    pltpu.sync_copy(pos_v.at[:, pl.ds(0, MAX_LEN)],
                    pos_hbm.at[pl.ds(base, RPW)])


def kernel(input_len, table):
    return _pe_kernel(input_len, table)
